# baseline (speedup 1.0000x reference)
import functools

import jax
import jax.numpy as jnp
from jax import lax
from jax.experimental import pallas as pl
from jax.experimental.pallas import tpu as pltpu
from jax.experimental.pallas import tpu_sc as plsc

_OFFSET = 16
_CLASSES = 8
_FEAT = _OFFSET * _CLASSES
_K = 64
_MAX_PTS = 1024
_NA = 128
_NV = _NA * _NA
_TOP_ROWS = _MAX_PTS // _NA
_TI = 32
_NTILES = _NA // _TI
_CAND = 2048
_CROWS = _CAND // _NA


def _score_phase(a_ref, bt_ref, wn_ref, bn_ref, wm_ref, bm_ref,
                 nmat_ref, mmat_ref):
    f_iota = jax.lax.broadcasted_iota(jnp.int32, (1, 1, _FEAT), 2)
    l_sh = f_iota >> 3
    c_id = f_iota & 7
    wn = wn_ref[...]
    bn = bn_ref[...]
    wm = wm_ref[...]
    bm = bm_ref[...]
    c0 = bt_ref[0, 0:1, :]
    c1 = bt_ref[0, 1:2, :]
    c2 = bt_ref[0, 2:3, :]

    def tile_body(s, _):
        rows = a_ref[0, pl.ds(s * _TI, _TI), :]
        f0 = rows[:, 0:1] + c0
        f1 = rows[:, 1:2] + c1
        f2 = rows[:, 2:3] + c2
        tok = ((f0[:, :, None] >> l_sh) & 1) \
            + 2 * ((f1[:, :, None] >> l_sh) & 1) \
            + 4 * ((f2[:, :, None] >> l_sh) & 1)
        feat = (tok == c_id).astype(jnp.float32)
        feat2 = feat.reshape(_TI * _NA, _FEAT)
        mm_n = jax.nn.relu(jnp.dot(feat2, wn) + bn)
        nmat_ref[pl.ds(s * _TI, _TI), :] = jnp.sum(
            mm_n.reshape(_TI, _NA, _K) / _K, axis=-1)
        mm_m = jax.nn.relu(jnp.dot(feat2, wm) + bm)
        mmat_ref[pl.ds(s * _TI, _TI), :] = jnp.sum(
            mm_m.reshape(_TI, _NA, _K) / _K, axis=-1)
        return 0

    jax.lax.fori_loop(0, _NTILES, tile_body, 0)


def _p_body(a_ref, bt_ref, wn_ref, bn_ref, wm_ref, bm_ref, p_ref,
            nmat_ref, mmat_ref):
    _score_phase(a_ref, bt_ref, wn_ref, bn_ref, wm_ref, bm_ref,
                 nmat_ref, mmat_ref)
    p_ref[0] = jax.lax.dot_general(
        nmat_ref[...], mmat_ref[...],
        dimension_numbers=(((1,), (1,)), ((), ())))


def _make_p_call():
    bsz = 8
    in_specs = [
        pl.BlockSpec((1, _NA, 3), lambda b: (b, 0, 0)),
        pl.BlockSpec((1, 3, _NA), lambda b: (b, 0, 0)),
        pl.BlockSpec((_FEAT, _K), lambda b: (0, 0)),
        pl.BlockSpec((1, _K), lambda b: (0, 0)),
        pl.BlockSpec((_FEAT, _K), lambda b: (0, 0)),
        pl.BlockSpec((1, _K), lambda b: (0, 0)),
    ]
    return pl.pallas_call(
        _p_body, grid=(bsz,), in_specs=in_specs,
        out_specs=pl.BlockSpec((1, _NA, _NA), lambda b: (b, 0, 0)),
        out_shape=jax.ShapeDtypeStruct((bsz, _NA, _NA), jnp.float32),
        scratch_shapes=[pltpu.VMEM((_NA, _NA), jnp.float32),
                        pltpu.VMEM((_NA, _NA), jnp.float32)])


def _sc_select_body(p_hbm, outv_hbm, outi_hbm, pv, hist, cv, ci):
    w = lax.axis_index("s") * 2 + lax.axis_index("c")

    @pl.when(w < 8)
    def _():
        pltpu.sync_copy(p_hbm.at[w], pv)
        lanes = lax.iota(jnp.int32, 16)
        ones = jnp.ones((16,), jnp.int32)

        prefix = jnp.int32(0)
        count_above = jnp.int32(0)
        for shift, nbits in ((23, 8), (15, 8), (7, 8), (0, 7)):
            nbins = 1 << nbits

            def zbody(i, _, nbins=nbins):
                hist[pl.ds(i * 16, 16)] = jnp.zeros((16,), jnp.int32)
                return 0
            lax.fori_loop(0, nbins, zbody, 0)

            top = shift + nbits

            def hbody(i, _, shift=shift, nbins=nbins, top=top,
                      prefix=prefix):
                x = plsc.bitcast(pv[pl.ds(i * 16, 16)], jnp.int32)
                sel = (x >> top) == prefix
                d = (x >> shift) & (nbins - 1)
                plsc.addupdate_scatter(hist, [(d << 4) | lanes], ones,
                                       mask=sel)
                return 0
            lax.fori_loop(0, _NV // 16, hbody, 0)

            def sbody(dd, carry, nbins=nbins):
                above, bstar, gtb = carry
                d = (nbins - 1) - dd
                tot = jnp.sum(hist[pl.ds(d * 16, 16)])
                hit = (above < _MAX_PTS) & (above + tot >= _MAX_PTS)
                bstar = jnp.where(hit, d, bstar)
                gtb = jnp.where(hit, above, gtb)
                return above + tot, bstar, gtb
            _, bstar, count_above = lax.fori_loop(
                0, nbins, sbody,
                (count_above, jnp.int32(0), jnp.int32(0)))
            prefix = (prefix << nbits) | bstar

        def ibody(i, _):
            cv[pl.ds(i * 16, 16)] = jnp.full((16,), -1.0, jnp.float32)
            ci[pl.ds(i * 16, 16)] = jnp.full((16,), _NV, jnp.int32)
            return 0
        lax.fori_loop(0, _CAND // 16, ibody, 0)

        def cbody(i, off, prefix=prefix):
            x = pv[pl.ds(i * 16, 16)]
            xi = plsc.bitcast(x, jnp.int32)
            m = xi >= prefix
            rank = plsc.cumsum(m.astype(jnp.int32)) - 1
            pos = jnp.minimum(off + rank, _CAND - 1)
            allow = m & ((off + rank) < _CAND)
            plsc.store_scatter(cv, [pos], x, mask=allow)
            plsc.store_scatter(ci, [pos], (i * 16) + lanes, mask=allow)
            return jnp.minimum(off + jnp.sum(m.astype(jnp.int32)),
                               jnp.int32(_CAND))
        lax.fori_loop(0, _NV // 16, cbody, jnp.int32(0))

        pltpu.sync_copy(cv, outv_hbm.at[w])
        pltpu.sync_copy(ci, outi_hbm.at[w])


def _make_sc_select():
    mesh = plsc.VectorSubcoreMesh(core_axis_name="c", subcore_axis_name="s")
    return pl.kernel(
        _sc_select_body,
        out_type=[jax.ShapeDtypeStruct((8, _CAND), jnp.float32),
                  jax.ShapeDtypeStruct((8, _CAND), jnp.int32)],
        mesh=mesh,
        compiler_params=pltpu.CompilerParams(needs_layout_passes=False),
        scratch_types=[pltpu.VMEM((_NV,), jnp.float32),
                       pltpu.VMEM((4096,), jnp.int32),
                       pltpu.VMEM((_CAND,), jnp.float32),
                       pltpu.VMEM((_CAND,), jnp.int32)])


def _static_roll(x, dist, rows):
    if dist < _NA:
        return jnp.concatenate([x[:, dist:], x[:, :dist]], axis=1)
    r = dist // _NA
    return jnp.concatenate([x[r:, :], x[:r, :]], axis=0)


def _static_stage(v, ix, flat, k, j, rows):
    up_mask = (flat & j) != 0
    want_larger = ((flat & k) == 0) == ((flat & j) == 0)
    nel = rows * _NA
    vd = _static_roll(v, j, rows)
    vu = _static_roll(v, nel - j if j >= _NA else _NA - j, rows)
    idn = _static_roll(ix, j, rows)
    iu = _static_roll(ix, nel - j if j >= _NA else _NA - j, rows)
    vp = jnp.where(up_mask, vu, vd)
    ip = jnp.where(up_mask, iu, idn)
    self_lt = (v < vp) | ((v == vp) & (ix > ip))
    take = self_lt == want_larger
    return jnp.where(take, vp, v), jnp.where(take, ip, ix)


def _sort_body(cv_ref, ci_ref, af_ref, vals_ref, sel_ref):
    v = cv_ref[0]
    ix = ci_ref[0]
    flat = jax.lax.broadcasted_iota(jnp.int32, (_CROWS, _NA), 0) * _NA \
        + jax.lax.broadcasted_iota(jnp.int32, (_CROWS, _NA), 1)
    for m in range(1, 12):
        k = 1 << m
        j = k // 2
        while j >= 1:
            v, ix = _static_stage(v, ix, flat, k, j, _CROWS)
            j //= 2

    v_top = v[:_TOP_ROWS, :]
    ix_top = ix[:_TOP_ROWS, :]
    idxmod = jax.lax.rem(ix_top, jnp.int32(384))

    af = af_ref[0]
    tv = jax.lax.broadcasted_iota(jnp.int32, (1, 1, 384), 2)
    selm = jnp.where(idxmod[:, :, None] == tv, af[None, :, :], 0.0)
    sel2 = jnp.sum(selm, axis=-1)

    pos = v_top > 0
    vals_ref[0] = jnp.where(pos, v_top, 0.0)
    sel_ref[0] = jnp.where(pos, sel2, 0.0)


def _make_sort_call():
    bsz = 8
    in_specs = [
        pl.BlockSpec((1, _CROWS, _NA), lambda b: (b, 0, 0)),
        pl.BlockSpec((1, _CROWS, _NA), lambda b: (b, 0, 0)),
        pl.BlockSpec((1, 1, 384), lambda b: (b, 0, 0)),
    ]
    out_specs = [
        pl.BlockSpec((1, _TOP_ROWS, _NA), lambda b: (b, 0, 0)),
        pl.BlockSpec((1, _TOP_ROWS, _NA), lambda b: (b, 0, 0)),
    ]
    out_shape = [
        jax.ShapeDtypeStruct((bsz, _TOP_ROWS, _NA), jnp.float32),
        jax.ShapeDtypeStruct((bsz, _TOP_ROWS, _NA), jnp.float32),
    ]
    return pl.pallas_call(_sort_body, grid=(bsz,), in_specs=in_specs,
                          out_specs=out_specs, out_shape=out_shape)


def kernel(a, b, W_n, b_n, W_m, b_m):
    bsz = a.shape[0]
    bt = jnp.transpose(b, (0, 2, 1))
    aflat = a.reshape(bsz, 1, 384).astype(jnp.float32)
    p = _make_p_call()(a, bt, W_n, b_n.reshape(1, _K), W_m,
                       b_m.reshape(1, _K))
    cv, ci = _make_sc_select()(p.reshape(bsz, _NV))
    vals, sel = _make_sort_call()(cv.reshape(bsz, _CROWS, _NA),
                                  ci.reshape(bsz, _CROWS, _NA), aflat)
    return vals.reshape(bsz, _MAX_PTS), sel.reshape(bsz, _MAX_PTS)


# speedup vs baseline: 1.1198x; 1.1198x over previous
import jax
import jax.numpy as jnp
from jax.experimental import pallas as pl
from jax.experimental.pallas import tpu as pltpu

_OFFSET = 16
_CLASSES = 8
_FEAT = _OFFSET * _CLASSES
_K = 64
_MAX_PTS = 1024
_NA = 128
_NV = _NA * _NA
_TOP_ROWS = _MAX_PTS // _NA
_TI = 32
_NTILES = _NA // _TI


def _score_phase(a_ref, bt_ref, wn_ref, bn_ref, wm_ref, bm_ref,
                 nmat_ref, mmat_ref):
    f_iota = jax.lax.broadcasted_iota(jnp.int32, (1, 1, _FEAT), 2)
    l_sh = f_iota >> 3
    c_id = f_iota & 7
    wn = wn_ref[...]
    bn = bn_ref[...]
    wm = wm_ref[...]
    bm = bm_ref[...]
    c0 = bt_ref[0, 0:1, :]
    c1 = bt_ref[0, 1:2, :]
    c2 = bt_ref[0, 2:3, :]

    def tile_body(s, _):
        rows = a_ref[0, pl.ds(s * _TI, _TI), :]
        f0 = rows[:, 0:1] + c0
        f1 = rows[:, 1:2] + c1
        f2 = rows[:, 2:3] + c2
        tok = ((f0[:, :, None] >> l_sh) & 1) \
            + 2 * ((f1[:, :, None] >> l_sh) & 1) \
            + 4 * ((f2[:, :, None] >> l_sh) & 1)
        feat = (tok == c_id).astype(jnp.float32)
        feat2 = feat.reshape(_TI * _NA, _FEAT)
        mm_n = jax.nn.relu(jnp.dot(feat2, wn) + bn)
        nmat_ref[pl.ds(s * _TI, _TI), :] = jnp.sum(
            mm_n.reshape(_TI, _NA, _K) / _K, axis=-1)
        mm_m = jax.nn.relu(jnp.dot(feat2, wm) + bm)
        mmat_ref[pl.ds(s * _TI, _TI), :] = jnp.sum(
            mm_m.reshape(_TI, _NA, _K) / _K, axis=-1)
        return 0

    jax.lax.fori_loop(0, _NTILES, tile_body, 0)


def _static_roll(x, dist, rows):
    if dist < _NA:
        return jnp.concatenate([x[:, dist:], x[:, :dist]], axis=1)
    r = dist // _NA
    return jnp.concatenate([x[r:, :], x[:r, :]], axis=0)


def _static_stage(v, ix, flat, k, j, rows):
    up_mask = (flat & j) != 0
    want_larger = ((flat & k) == 0) == ((flat & j) == 0)
    nel = rows * _NA
    vd = _static_roll(v, j, rows)
    vu = _static_roll(v, nel - j if j >= _NA else _NA - j, rows)
    idn = _static_roll(ix, j, rows)
    iu = _static_roll(ix, nel - j if j >= _NA else _NA - j, rows)
    vp = jnp.where(up_mask, vu, vd)
    ip = jnp.where(up_mask, iu, idn)
    self_lt = (v < vp) | ((v == vp) & (ix > ip))
    take = self_lt == want_larger
    return jnp.where(take, vp, v), jnp.where(take, ip, ix)


def _sort_stage(flat, v_ref, ix_ref, k, j, r, axis):
    v = v_ref[...]
    ix = ix_ref[...]
    up_mask = (flat & j) != 0
    want_larger = ((flat & k) == 0) == ((flat & j) == 0)
    vu = pltpu.roll(v, r, axis)
    vd = pltpu.roll(v, _NA - r, axis)
    iu = pltpu.roll(ix, r, axis)
    idn = pltpu.roll(ix, _NA - r, axis)
    vp = jnp.where(up_mask, vu, vd)
    ip = jnp.where(up_mask, iu, idn)
    self_lt = (v < vp) | ((v == vp) & (ix > ip))
    take = self_lt == want_larger
    v_ref[...] = jnp.where(take, vp, v)
    ix_ref[...] = jnp.where(take, ip, ix)


def _decoder_body(a_ref, bt_ref, wn_ref, bn_ref, wm_ref,
                  bm_ref, af_ref, vals_ref, sel_ref,
                  nmat_ref, mmat_ref, v_ref, ix_ref):
    _score_phase(a_ref, bt_ref, wn_ref, bn_ref, wm_ref, bm_ref,
                 nmat_ref, mmat_ref)

    row_i = jax.lax.broadcasted_iota(jnp.int32, (_NA, _NA), 0)
    col_i = jax.lax.broadcasted_iota(jnp.int32, (_NA, _NA), 1)
    flat = row_i * _NA + col_i

    v_ref[...] = jax.lax.dot_general(
        nmat_ref[...], mmat_ref[...],
        dimension_numbers=(((1,), (1,)), ((), ())))
    ix_ref[...] = flat

    for m in range(1, 11):
        k = 1 << m
        nrow = max(0, m - 7)
        if nrow > 0:
            def row_body(s, _, m=m, k=k):
                t = (m - 1) - s
                j = jnp.int32(1) << t
                r = jnp.int32(1) << (t - 7)
                _sort_stage(flat, v_ref, ix_ref, k, j, r, 0)
                return 0
            jax.lax.fori_loop(0, nrow, row_body, 0)

        nlane = min(m, 7)
        lane_t0 = min(m - 1, 6)
        def lane_body(s, _, k=k, lane_t0=lane_t0):
            t = lane_t0 - s
            j = jnp.int32(1) << t
            _sort_stage(flat, v_ref, ix_ref, k, j, j, 1)
            return 0
        jax.lax.fori_loop(0, nlane, lane_body, 0)

    v = v_ref[...]
    ix = ix_ref[...]
    rows = _NA
    while rows > _TOP_ROWS:
        g = rows // 16
        v4 = v.reshape(g, 16, _NA)
        i4 = ix.reshape(g, 16, _NA)
        av, bv = v4[:, :8, :], v4[:, 8:, :]
        ai, bi = i4[:, :8, :], i4[:, 8:, :]
        lt = (av < bv) | ((av == bv) & (ai > bi))
        rows = rows // 2
        v = jnp.where(lt, bv, av).reshape(rows, _NA)
        ix = jnp.where(lt, bi, ai).reshape(rows, _NA)
        kk = 1024 if rows > _TOP_ROWS else 2048
        fl = jax.lax.broadcasted_iota(jnp.int32, (rows, _NA), 0) * _NA \
            + jax.lax.broadcasted_iota(jnp.int32, (rows, _NA), 1)
        j = 512
        while j >= 1:
            v, ix = _static_stage(v, ix, fl, kk, j, rows)
            j //= 2

    v_top = v
    ix_top = ix
    idxmod = jax.lax.rem(ix_top, jnp.int32(384))

    af = af_ref[0]
    tv = jax.lax.broadcasted_iota(jnp.int32, (1, 1, 384), 2)
    selm = jnp.where(idxmod[:, :, None] == tv, af[None, :, :], 0.0)
    sel2 = jnp.sum(selm, axis=-1)

    pos = v_top > 0
    vals_ref[0] = jnp.where(pos, v_top, 0.0)
    sel_ref[0] = jnp.where(pos, sel2, 0.0)


def _make_call(interpret=False):
    bsz = 8
    grid = (bsz,)
    in_specs = [
        pl.BlockSpec((1, _NA, 3), lambda b: (b, 0, 0)),
        pl.BlockSpec((1, 3, _NA), lambda b: (b, 0, 0)),
        pl.BlockSpec((_FEAT, _K), lambda b: (0, 0)),
        pl.BlockSpec((1, _K), lambda b: (0, 0)),
        pl.BlockSpec((_FEAT, _K), lambda b: (0, 0)),
        pl.BlockSpec((1, _K), lambda b: (0, 0)),
        pl.BlockSpec((1, 1, 384), lambda b: (b, 0, 0)),
    ]
    out_specs = [
        pl.BlockSpec((1, _TOP_ROWS, _NA), lambda b: (b, 0, 0)),
        pl.BlockSpec((1, _TOP_ROWS, _NA), lambda b: (b, 0, 0)),
    ]
    out_shape = [
        jax.ShapeDtypeStruct((bsz, _TOP_ROWS, _NA), jnp.float32),
        jax.ShapeDtypeStruct((bsz, _TOP_ROWS, _NA), jnp.float32),
    ]
    scratch_shapes = [
        pltpu.VMEM((_NA, _NA), jnp.float32),
        pltpu.VMEM((_NA, _NA), jnp.float32),
        pltpu.VMEM((_NA, _NA), jnp.float32),
        pltpu.VMEM((_NA, _NA), jnp.int32),
    ]
    return pl.pallas_call(_decoder_body, grid=grid, in_specs=in_specs,
                          out_specs=out_specs, out_shape=out_shape,
                          scratch_shapes=scratch_shapes,
                          interpret=interpret)


def kernel(a, b, W_n, b_n, W_m, b_m):
    bsz = a.shape[0]
    bt = jnp.transpose(b, (0, 2, 1))
    aflat = a.reshape(bsz, 1, 384).astype(jnp.float32)
    call = _make_call()
    vals, sel = call(a, bt, W_n, b_n.reshape(1, _K), W_m,
                     b_m.reshape(1, _K), aflat)
    return vals.reshape(bsz, _MAX_PTS), sel.reshape(bsz, _MAX_PTS)


# speedup vs baseline: 1.2303x; 1.0987x over previous
import jax
import jax.numpy as jnp
from jax.experimental import pallas as pl
from jax.experimental.pallas import tpu as pltpu

_OFFSET = 16
_CLASSES = 8
_FEAT = _OFFSET * _CLASSES
_K = 64
_MAX_PTS = 1024
_NA = 128
_NV = _NA * _NA
_TOP_ROWS = _MAX_PTS // _NA
_TI = 32
_NTILES = _NA // _TI


def _score_phase(a_ref, bt_ref, wn_ref, bn_ref, wm_ref, bm_ref,
                 nmat_ref, mmat_ref):
    f_iota = jax.lax.broadcasted_iota(jnp.int32, (1, 1, _FEAT), 2)
    l_sh = f_iota >> 3
    c_id = f_iota & 7
    wn = wn_ref[...]
    bn = bn_ref[...]
    wm = wm_ref[...]
    bm = bm_ref[...]
    c0 = bt_ref[0, 0:1, :]
    c1 = bt_ref[0, 1:2, :]
    c2 = bt_ref[0, 2:3, :]

    def tile_body(s, _):
        rows = a_ref[0, pl.ds(s * _TI, _TI), :]
        f0 = rows[:, 0:1] + c0
        f1 = rows[:, 1:2] + c1
        f2 = rows[:, 2:3] + c2
        tok = ((f0[:, :, None] >> l_sh) & 1) \
            + 2 * ((f1[:, :, None] >> l_sh) & 1) \
            + 4 * ((f2[:, :, None] >> l_sh) & 1)
        feat = (tok == c_id).astype(jnp.float32)
        feat2 = feat.reshape(_TI * _NA, _FEAT)
        mm_n = jax.nn.relu(jnp.dot(feat2, wn) + bn)
        nmat_ref[pl.ds(s * _TI, _TI), :] = jnp.sum(
            mm_n.reshape(_TI, _NA, _K) / _K, axis=-1)
        mm_m = jax.nn.relu(jnp.dot(feat2, wm) + bm)
        mmat_ref[pl.ds(s * _TI, _TI), :] = jnp.sum(
            mm_m.reshape(_TI, _NA, _K) / _K, axis=-1)
        return 0

    jax.lax.fori_loop(0, _NTILES, tile_body, 0)


def _static_roll(x, dist, rows):
    if dist < _NA:
        return jnp.concatenate([x[:, dist:], x[:, :dist]], axis=1)
    r = dist // _NA
    return jnp.concatenate([x[r:, :], x[:r, :]], axis=0)


def _static_stage(v, ix, flat, k, j, rows):
    up_mask = (flat & j) != 0
    want_larger = ((flat & k) == 0) == ((flat & j) == 0)
    nel = rows * _NA
    vd = _static_roll(v, j, rows)
    vu = _static_roll(v, nel - j if j >= _NA else _NA - j, rows)
    idn = _static_roll(ix, j, rows)
    iu = _static_roll(ix, nel - j if j >= _NA else _NA - j, rows)
    vp = jnp.where(up_mask, vu, vd)
    ip = jnp.where(up_mask, iu, idn)
    self_lt = (v < vp) | ((v == vp) & (ix > ip))
    take = self_lt == want_larger
    return jnp.where(take, vp, v), jnp.where(take, ip, ix)


def _sort_stage(flat, v_ref, ix_ref, k, j, r, axis):
    v = v_ref[...]
    ix = ix_ref[...]
    up_mask = (flat & j) != 0
    want_larger = ((flat & k) == 0) == ((flat & j) == 0)
    vu = pltpu.roll(v, r, axis)
    vd = pltpu.roll(v, _NA - r, axis)
    iu = pltpu.roll(ix, r, axis)
    idn = pltpu.roll(ix, _NA - r, axis)
    vp = jnp.where(up_mask, vu, vd)
    ip = jnp.where(up_mask, iu, idn)
    self_lt = (v < vp) | ((v == vp) & (ix > ip))
    take = self_lt == want_larger
    v_ref[...] = jnp.where(take, vp, v)
    ix_ref[...] = jnp.where(take, ip, ix)


def _decoder_body(a_ref, bt_ref, wn_ref, bn_ref, wm_ref,
                  bm_ref, af_ref, vals_ref, sel_ref,
                  nmat_ref, mmat_ref, v_ref, ix_ref):
    _score_phase(a_ref, bt_ref, wn_ref, bn_ref, wm_ref, bm_ref,
                 nmat_ref, mmat_ref)

    row_i = jax.lax.broadcasted_iota(jnp.int32, (_NA, _NA), 0)
    col_i = jax.lax.broadcasted_iota(jnp.int32, (_NA, _NA), 1)
    flat = row_i * _NA + col_i

    v_ref[...] = jax.lax.dot_general(
        nmat_ref[...], mmat_ref[...],
        dimension_numbers=(((1,), (1,)), ((), ())))
    ix_ref[...] = flat

    v = v_ref[...]
    ix = ix_ref[...]
    for m in range(1, 8):
        k = 1 << m
        j = k // 2
        while j >= 1:
            v, ix = _static_stage(v, ix, flat, k, j, _NA)
            j //= 2
    v_ref[...] = v
    ix_ref[...] = ix

    for m in range(8, 11):
        k = 1 << m
        nrow = max(0, m - 7)
        if nrow > 0:
            def row_body(s, _, m=m, k=k):
                t = (m - 1) - s
                j = jnp.int32(1) << t
                r = jnp.int32(1) << (t - 7)
                _sort_stage(flat, v_ref, ix_ref, k, j, r, 0)
                return 0
            jax.lax.fori_loop(0, nrow, row_body, 0)

        nlane = min(m, 7)
        lane_t0 = min(m - 1, 6)
        def lane_body(s, _, k=k, lane_t0=lane_t0):
            t = lane_t0 - s
            j = jnp.int32(1) << t
            _sort_stage(flat, v_ref, ix_ref, k, j, j, 1)
            return 0
        jax.lax.fori_loop(0, nlane, lane_body, 0)

    v = v_ref[...]
    ix = ix_ref[...]
    rows = _NA
    while rows > _TOP_ROWS:
        g = rows // 16
        v4 = v.reshape(g, 16, _NA)
        i4 = ix.reshape(g, 16, _NA)
        av, bv = v4[:, :8, :], v4[:, 8:, :]
        ai, bi = i4[:, :8, :], i4[:, 8:, :]
        lt = (av < bv) | ((av == bv) & (ai > bi))
        rows = rows // 2
        v = jnp.where(lt, bv, av).reshape(rows, _NA)
        ix = jnp.where(lt, bi, ai).reshape(rows, _NA)
        kk = 1024 if rows > _TOP_ROWS else 2048
        fl = jax.lax.broadcasted_iota(jnp.int32, (rows, _NA), 0) * _NA \
            + jax.lax.broadcasted_iota(jnp.int32, (rows, _NA), 1)
        j = 512
        while j >= 1:
            v, ix = _static_stage(v, ix, fl, kk, j, rows)
            j //= 2

    v_top = v
    ix_top = ix
    idxmod = jax.lax.rem(ix_top, jnp.int32(384))

    af = af_ref[0]
    tv = jax.lax.broadcasted_iota(jnp.int32, (1, 1, 384), 2)
    selm = jnp.where(idxmod[:, :, None] == tv, af[None, :, :], 0.0)
    sel2 = jnp.sum(selm, axis=-1)

    pos = v_top > 0
    vals_ref[0] = jnp.where(pos, v_top, 0.0)
    sel_ref[0] = jnp.where(pos, sel2, 0.0)


def _make_call(interpret=False):
    bsz = 8
    grid = (bsz,)
    in_specs = [
        pl.BlockSpec((1, _NA, 3), lambda b: (b, 0, 0)),
        pl.BlockSpec((1, 3, _NA), lambda b: (b, 0, 0)),
        pl.BlockSpec((_FEAT, _K), lambda b: (0, 0)),
        pl.BlockSpec((1, _K), lambda b: (0, 0)),
        pl.BlockSpec((_FEAT, _K), lambda b: (0, 0)),
        pl.BlockSpec((1, _K), lambda b: (0, 0)),
        pl.BlockSpec((1, 1, 384), lambda b: (b, 0, 0)),
    ]
    out_specs = [
        pl.BlockSpec((1, _TOP_ROWS, _NA), lambda b: (b, 0, 0)),
        pl.BlockSpec((1, _TOP_ROWS, _NA), lambda b: (b, 0, 0)),
    ]
    out_shape = [
        jax.ShapeDtypeStruct((bsz, _TOP_ROWS, _NA), jnp.float32),
        jax.ShapeDtypeStruct((bsz, _TOP_ROWS, _NA), jnp.float32),
    ]
    scratch_shapes = [
        pltpu.VMEM((_NA, _NA), jnp.float32),
        pltpu.VMEM((_NA, _NA), jnp.float32),
        pltpu.VMEM((_NA, _NA), jnp.float32),
        pltpu.VMEM((_NA, _NA), jnp.int32),
    ]
    return pl.pallas_call(_decoder_body, grid=grid, in_specs=in_specs,
                          out_specs=out_specs, out_shape=out_shape,
                          scratch_shapes=scratch_shapes,
                          interpret=interpret)


def kernel(a, b, W_n, b_n, W_m, b_m):
    bsz = a.shape[0]
    bt = jnp.transpose(b, (0, 2, 1))
    aflat = a.reshape(bsz, 1, 384).astype(jnp.float32)
    call = _make_call()
    vals, sel = call(a, bt, W_n, b_n.reshape(1, _K), W_m,
                     b_m.reshape(1, _K), aflat)
    return vals.reshape(bsz, _MAX_PTS), sel.reshape(bsz, _MAX_PTS)


# speedup vs baseline: 1.2790x; 1.0396x over previous
import jax
import jax.numpy as jnp
from jax.experimental import pallas as pl
from jax.experimental.pallas import tpu as pltpu

_OFFSET = 16
_CLASSES = 8
_FEAT = _OFFSET * _CLASSES
_K = 64
_MAX_PTS = 1024
_NA = 128
_NV = _NA * _NA
_TOP_ROWS = _MAX_PTS // _NA
_TI = 32
_NTILES = _NA // _TI


def _score_phase(a_ref, bt_ref, wn_ref, bn_ref, wm_ref, bm_ref,
                 nmat_ref, mmat_ref):
    f_iota = jax.lax.broadcasted_iota(jnp.int32, (1, 1, _FEAT), 2)
    l_sh = f_iota >> 3
    c_id = f_iota & 7
    wn = wn_ref[...]
    bn = bn_ref[...]
    wm = wm_ref[...]
    bm = bm_ref[...]
    c0 = bt_ref[0, 0:1, :]
    c1 = bt_ref[0, 1:2, :]
    c2 = bt_ref[0, 2:3, :]

    def tile_body(s, _):
        rows = a_ref[0, pl.ds(s * _TI, _TI), :]
        f0 = rows[:, 0:1] + c0
        f1 = rows[:, 1:2] + c1
        f2 = rows[:, 2:3] + c2
        tok = ((f0[:, :, None] >> l_sh) & 1) \
            + 2 * ((f1[:, :, None] >> l_sh) & 1) \
            + 4 * ((f2[:, :, None] >> l_sh) & 1)
        feat = (tok == c_id).astype(jnp.float32)
        feat2 = feat.reshape(_TI * _NA, _FEAT)
        mm_n = jax.nn.relu(jnp.dot(feat2, wn) + bn)
        nmat_ref[pl.ds(s * _TI, _TI), :] = jnp.sum(
            mm_n.reshape(_TI, _NA, _K) / _K, axis=-1)
        mm_m = jax.nn.relu(jnp.dot(feat2, wm) + bm)
        mmat_ref[pl.ds(s * _TI, _TI), :] = jnp.sum(
            mm_m.reshape(_TI, _NA, _K) / _K, axis=-1)
        return 0

    jax.lax.fori_loop(0, _NTILES, tile_body, 0)


def _static_roll(x, dist, rows):
    if dist < _NA:
        return jnp.concatenate([x[:, dist:], x[:, :dist]], axis=1)
    r = dist // _NA
    return jnp.concatenate([x[r:, :], x[:r, :]], axis=0)


def _static_stage(v, ix, flat, k, j, rows):
    up_mask = (flat & j) != 0
    want_larger = ((flat & k) == 0) == ((flat & j) == 0)
    nel = rows * _NA
    vd = _static_roll(v, j, rows)
    vu = _static_roll(v, nel - j if j >= _NA else _NA - j, rows)
    idn = _static_roll(ix, j, rows)
    iu = _static_roll(ix, nel - j if j >= _NA else _NA - j, rows)
    vp = jnp.where(up_mask, vu, vd)
    ip = jnp.where(up_mask, iu, idn)
    self_lt = (v < vp) | ((v == vp) & (ix > ip))
    take = self_lt == want_larger
    return jnp.where(take, vp, v), jnp.where(take, ip, ix)


def _sort_stage(flat, v_ref, ix_ref, k, j, r, axis):
    v = v_ref[...]
    ix = ix_ref[...]
    up_mask = (flat & j) != 0
    want_larger = ((flat & k) == 0) == ((flat & j) == 0)
    vu = pltpu.roll(v, r, axis)
    vd = pltpu.roll(v, _NA - r, axis)
    iu = pltpu.roll(ix, r, axis)
    idn = pltpu.roll(ix, _NA - r, axis)
    vp = jnp.where(up_mask, vu, vd)
    ip = jnp.where(up_mask, iu, idn)
    self_lt = (v < vp) | ((v == vp) & (ix > ip))
    take = self_lt == want_larger
    v_ref[...] = jnp.where(take, vp, v)
    ix_ref[...] = jnp.where(take, ip, ix)


def _decoder_body(a_ref, bt_ref, wn_ref, bn_ref, wm_ref,
                  bm_ref, af_ref, vals_ref, sel_ref,
                  nmat_ref, mmat_ref, v_ref, ix_ref):
    _score_phase(a_ref, bt_ref, wn_ref, bn_ref, wm_ref, bm_ref,
                 nmat_ref, mmat_ref)

    row_i = jax.lax.broadcasted_iota(jnp.int32, (_NA, _NA), 0)
    col_i = jax.lax.broadcasted_iota(jnp.int32, (_NA, _NA), 1)
    flat = row_i * _NA + col_i

    v_ref[...] = jax.lax.dot_general(
        nmat_ref[...], mmat_ref[...],
        dimension_numbers=(((1,), (1,)), ((), ())))
    ix_ref[...] = flat

    v = v_ref[...]
    ix = ix_ref[...]
    for m in range(1, 8):
        k = 1 << m
        j = k // 2
        while j >= 1:
            v, ix = _static_stage(v, ix, flat, k, j, _NA)
            j //= 2
    v_ref[...] = v
    ix_ref[...] = ix

    for m in range(8, 11):
        k = 1 << m
        nrow = max(0, m - 7)
        if nrow > 0:
            def row_body(s, _, m=m, k=k):
                t = (m - 1) - s
                j = jnp.int32(1) << t
                r = jnp.int32(1) << (t - 7)
                _sort_stage(flat, v_ref, ix_ref, k, j, r, 0)
                return 0
            jax.lax.fori_loop(0, nrow, row_body, 0)

        v = v_ref[...]
        ix = ix_ref[...]
        j = 64
        while j >= 1:
            v, ix = _static_stage(v, ix, flat, k, j, _NA)
            j //= 2
        v_ref[...] = v
        ix_ref[...] = ix

    v = v_ref[...]
    ix = ix_ref[...]
    rows = _NA
    while rows > _TOP_ROWS:
        g = rows // 16
        v4 = v.reshape(g, 16, _NA)
        i4 = ix.reshape(g, 16, _NA)
        av, bv = v4[:, :8, :], v4[:, 8:, :]
        ai, bi = i4[:, :8, :], i4[:, 8:, :]
        lt = (av < bv) | ((av == bv) & (ai > bi))
        rows = rows // 2
        v = jnp.where(lt, bv, av).reshape(rows, _NA)
        ix = jnp.where(lt, bi, ai).reshape(rows, _NA)
        kk = 1024 if rows > _TOP_ROWS else 2048
        fl = jax.lax.broadcasted_iota(jnp.int32, (rows, _NA), 0) * _NA \
            + jax.lax.broadcasted_iota(jnp.int32, (rows, _NA), 1)
        j = 512
        while j >= 1:
            v, ix = _static_stage(v, ix, fl, kk, j, rows)
            j //= 2

    v_top = v
    ix_top = ix
    idxmod = jax.lax.rem(ix_top, jnp.int32(384))

    af = af_ref[0]
    tv = jax.lax.broadcasted_iota(jnp.int32, (1, 1, 384), 2)
    selm = jnp.where(idxmod[:, :, None] == tv, af[None, :, :], 0.0)
    sel2 = jnp.sum(selm, axis=-1)

    pos = v_top > 0
    vals_ref[0] = jnp.where(pos, v_top, 0.0)
    sel_ref[0] = jnp.where(pos, sel2, 0.0)


def _make_call(interpret=False):
    bsz = 8
    grid = (bsz,)
    in_specs = [
        pl.BlockSpec((1, _NA, 3), lambda b: (b, 0, 0)),
        pl.BlockSpec((1, 3, _NA), lambda b: (b, 0, 0)),
        pl.BlockSpec((_FEAT, _K), lambda b: (0, 0)),
        pl.BlockSpec((1, _K), lambda b: (0, 0)),
        pl.BlockSpec((_FEAT, _K), lambda b: (0, 0)),
        pl.BlockSpec((1, _K), lambda b: (0, 0)),
        pl.BlockSpec((1, 1, 384), lambda b: (b, 0, 0)),
    ]
    out_specs = [
        pl.BlockSpec((1, _TOP_ROWS, _NA), lambda b: (b, 0, 0)),
        pl.BlockSpec((1, _TOP_ROWS, _NA), lambda b: (b, 0, 0)),
    ]
    out_shape = [
        jax.ShapeDtypeStruct((bsz, _TOP_ROWS, _NA), jnp.float32),
        jax.ShapeDtypeStruct((bsz, _TOP_ROWS, _NA), jnp.float32),
    ]
    scratch_shapes = [
        pltpu.VMEM((_NA, _NA), jnp.float32),
        pltpu.VMEM((_NA, _NA), jnp.float32),
        pltpu.VMEM((_NA, _NA), jnp.float32),
        pltpu.VMEM((_NA, _NA), jnp.int32),
    ]
    return pl.pallas_call(_decoder_body, grid=grid, in_specs=in_specs,
                          out_specs=out_specs, out_shape=out_shape,
                          scratch_shapes=scratch_shapes,
                          interpret=interpret)


def kernel(a, b, W_n, b_n, W_m, b_m):
    bsz = a.shape[0]
    bt = jnp.transpose(b, (0, 2, 1))
    aflat = a.reshape(bsz, 1, 384).astype(jnp.float32)
    call = _make_call()
    vals, sel = call(a, bt, W_n, b_n.reshape(1, _K), W_m,
                     b_m.reshape(1, _K), aflat)
    return vals.reshape(bsz, _MAX_PTS), sel.reshape(bsz, _MAX_PTS)
